# baseline (speedup 1.0000x reference)
import functools

import jax
import jax.numpy as jnp
from jax import lax
from jax.experimental import pallas as pl
from jax.experimental.pallas import tpu as pltpu
from jax.experimental.pallas import tpu_sc as plsc

_NC = 2
_NS = 16
_NW = _NC * _NS
_LANES = 16
_CHUNK = 128


def _make_sc_dots(B, D, K):
    rows_per_w = B // _NW
    n_chunks = rows_per_w // _CHUNK
    n_groups = _CHUNK // _LANES
    mesh = plsc.VectorSubcoreMesh(core_axis_name="c", subcore_axis_name="s")

    @functools.partial(
        pl.kernel,
        mesh=mesh,
        compiler_params=pltpu.CompilerParams(
            use_tc_tiling_on_sc=False, needs_layout_passes=False
        ),
        out_type=jax.ShapeDtypeStruct((16, B), jnp.float32),
        scratch_types=(
            [pltpu.VMEM((K, _CHUNK), jnp.int32)]
            + [pltpu.VMEM((_CHUNK, D), jnp.float32) for _ in range(K)]
            + [pltpu.VMEM((16, _CHUNK), jnp.float32)]
            + [pltpu.SemaphoreType.DMA]
        ),
    )
    def sc(idx_hbm, emb_hbm, w_hbm, out_hbm, idx_v, *rest):
        bufs = rest[:K]
        logit_v = rest[K]
        sem = rest[K + 1]
        wid = lax.axis_index("s") * _NC + lax.axis_index("c")
        base_w = wid * rows_per_w

        def chunk_body(ci, carry):
            base = base_w + ci * _CHUNK
            pltpu.sync_copy(idx_hbm.at[:, pl.ds(base, _CHUNK)], idx_v)
            handles = [pltpu.async_copy(emb_hbm.at[idx_v.at[0]], bufs[0], sem)]
            for j in range(1, K):
                handles.append(
                    pltpu.async_copy(w_hbm.at[idx_v.at[j]], bufs[j], sem)
                )
            for h in handles:
                h.wait()
            for g in range(n_groups):
                row_idx = (
                    lax.broadcasted_iota(jnp.int32, (_LANES,), 0) + g * _LANES
                )

                def dbody(d, accs):
                    col = lax.broadcast(d, (_LANES,))
                    e = plsc.load_gather(bufs[0], [row_idx, col])
                    return tuple(
                        accs[j] + e * plsc.load_gather(bufs[j + 1], [row_idx, col])
                        for j in range(K - 1)
                    )

                accs = lax.fori_loop(
                    0,
                    D,
                    dbody,
                    tuple(jnp.zeros((_LANES,), jnp.float32) for _ in range(K - 1)),
                )
                for j in range(K - 1):
                    logit_v[j, pl.ds(g * _LANES, _LANES)] = accs[j]
            pltpu.sync_copy(logit_v, out_hbm.at[:, pl.ds(base, _CHUNK)])
            return carry

        lax.fori_loop(0, n_chunks, chunk_body, 0)

    return sc


def _make_tc_loss(B, NEG):
    def body(l_ref, o_ref):
        z = l_ref[...]
        row = lax.broadcasted_iota(jnp.int32, z.shape, 0)
        zz = jnp.where(row == 0, z, -z)
        ls = jnp.log(jax.nn.sigmoid(zz))
        ls = jnp.where(row <= NEG, ls, 0.0)
        total = jnp.sum(ls, axis=1, keepdims=True)
        o_ref[...] = -jnp.sum(total, axis=0, keepdims=True) / jnp.float32(B)

    return pl.pallas_call(
        body,
        out_shape=jax.ShapeDtypeStruct((1, 1), jnp.float32),
    )


def kernel(x, targets, negatives, emb_table, out_weight):
    B = x.shape[0]
    NEG = negatives.shape[1]
    D = emb_table.shape[1]
    all_idx = jnp.concatenate(
        [
            x.astype(jnp.int32)[None, :],
            targets.astype(jnp.int32)[None, :],
            negatives.astype(jnp.int32).T,
        ],
        axis=0,
    )
    logits = _make_sc_dots(B, D, NEG + 2)(all_idx, emb_table, out_weight)
    loss = _make_tc_loss(B, NEG)(logits)
    return loss[0, 0]
